# Initial kernel scaffold; baseline (speedup 1.0000x reference)
import jax
import jax.numpy as jnp
from jax.experimental import pallas as pl


def kernel(x, sw1, sb1, sw2, sb2, sw3, sb3, mw1, mb1, mw2, mb2, mw3, mb3, mw4, mb4):
    raise NotImplementedError("write your pallas kernel here")


# speedup vs baseline: 1.0726x; 1.0726x over previous
import functools

import jax
import jax.numpy as jnp
from jax.experimental import pallas as pl
from jax.experimental.pallas import tpu as pltpu

_SLOPE = 0.01
_B = 4
_D = 128
_HW = _D * _D
_S = 2
_P = _HW // _S
_DC = 8
_NJ = _D // _DC


def _lrelu(x):
    return jnp.where(x >= 0, x, _SLOPE * x)


def _body(x_ref, sw1_ref, sb1_ref, sw2_ref, sb2_ref, sw3_ref, sb3_ref,
          mw1f_ref, mw1p_ref, mb1_ref, mw2_ref, mb2_ref, mw3_ref, mb3_ref,
          mw4_ref, mb4_ref, out_ref, base_scr):
    j = pl.program_id(2)

    @pl.when(j == 0)
    def _stage():
        xb = x_ref[0]
        f = _lrelu(jnp.dot(sw1_ref[...], xb,
                           preferred_element_type=jnp.float32) + sb1_ref[...])
        f = _lrelu(jnp.dot(sw2_ref[...], f,
                           preferred_element_type=jnp.float32) + sb2_ref[...])
        f = jnp.dot(sw3_ref[...], f,
                    preferred_element_type=jnp.float32) + sb3_ref[...]
        base_scr[...] = jnp.dot(mw1f_ref[...], f,
                                preferred_element_type=jnp.float32) + mb1_ref[...]

    drow = jax.lax.broadcasted_iota(jnp.int32, (1, _DC, 1), 1).astype(jnp.float32)
    pt = (jnp.float32(j * _DC) + drow) * (2.0 / (_D - 1)) - 1.0
    coef = mw1p_ref[...] * pt

    h = _lrelu(base_scr[...][:, None, :] + coef)
    h = _lrelu(jnp.einsum('qj,jpm->qpm', mw2_ref[...], h,
                          preferred_element_type=jnp.float32) + mb2_ref[...])
    h = _lrelu(jnp.einsum('qj,jpm->qpm', mw3_ref[...], h,
                          preferred_element_type=jnp.float32) + mb3_ref[...])
    o = jnp.einsum('qj,jpm->qpm', mw4_ref[...], h,
                   preferred_element_type=jnp.float32) + mb4_ref[...]
    out_ref[...] = jax.nn.sigmoid(o)


@jax.jit
def kernel(x, sw1, sb1, sw2, sb2, sw3, sb3,
           mw1, mb1, mw2, mb2, mw3, mb3, mw4, mb4):
    xr = x.reshape(_B, _D, _HW)

    def wspec(shape):
        return pl.BlockSpec(shape, lambda b, s, j: (0,) * len(shape))

    grid = (_B, _S, _NJ)
    out = pl.pallas_call(
        _body,
        grid=grid,
        in_specs=[
            pl.BlockSpec((1, _D, _P), lambda b, s, j: (b, 0, s)),
            wspec((64, 128)), wspec((64, 1)),
            wspec((32, 64)), wspec((32, 1)),
            wspec((16, 32)), wspec((16, 1)),
            wspec((32, 16)), wspec((32, 1, 1)), wspec((32, 1)),
            wspec((32, 32)), wspec((32, 1, 1)),
            wspec((16, 32)), wspec((16, 1, 1)),
            wspec((1, 16)), wspec((1, 1, 1)),
        ],
        out_specs=pl.BlockSpec((1, _DC, _P), lambda b, s, j: (b, j, s)),
        out_shape=jax.ShapeDtypeStruct((_B, _D, _HW), jnp.float32),
        scratch_shapes=[pltpu.VMEM((32, _P), jnp.float32)],
        compiler_params=pltpu.CompilerParams(
            dimension_semantics=("parallel", "parallel", "arbitrary"),
            vmem_limit_bytes=100 * 1024 * 1024,
        ),
    )(xr, sw1, sb1.reshape(64, 1), sw2, sb2.reshape(32, 1),
      sw3, sb3.reshape(16, 1),
      mw1[:, :16], mw1[:, 16:].reshape(32, 1, 1), mb1.reshape(32, 1),
      mw2, mb2.reshape(32, 1, 1), mw3, mb3.reshape(16, 1, 1),
      mw4, mb4.reshape(1, 1, 1))
    return out.reshape(_B, _D, _D, _D)


# speedup vs baseline: 3.6634x; 3.4154x over previous
import jax
import jax.numpy as jnp
from jax.experimental import pallas as pl
from jax.experimental.pallas import tpu as pltpu

_SLOPE = 0.01
_B = 4
_D = 128
_HW = _D * _D
_S = 2
_P = _HW // _S
_DC = 8
_NJ = _D // _DC
_N = _DC * _P


def _lrelu(x):
    return jnp.maximum(x, _SLOPE * x)


def _body(x_ref, sw1_ref, sb1_ref, sw2_ref, sb2_ref, sw3_ref, sb3_ref,
          mw1f_ref, mw1p_ref, mb1_ref, mw2_ref, mb2_ref, mw3_ref, mb3_ref,
          mw4_ref, mb4_ref, out_ref, base_scr):
    j = pl.program_id(2)

    @pl.when(j == 0)
    def _stage():
        xb = x_ref[0]
        f = _lrelu(jnp.dot(sw1_ref[...], xb,
                           preferred_element_type=jnp.float32) + sb1_ref[...])
        f = _lrelu(jnp.dot(sw2_ref[...], f,
                           preferred_element_type=jnp.float32) + sb2_ref[...])
        f = jnp.dot(sw3_ref[...], f,
                    preferred_element_type=jnp.float32) + sb3_ref[...]
        base_scr[...] = jnp.dot(mw1f_ref[...], f,
                                preferred_element_type=jnp.float32) + mb1_ref[...]

    base = base_scr[...].astype(jnp.bfloat16)
    w1p = mw1p_ref[...]
    pieces = []
    for d in range(_DC):
        pt = (j * _DC + d).astype(jnp.float32) * (2.0 / (_D - 1)) - 1.0
        pieces.append(base + (w1p * pt).astype(jnp.bfloat16))
    h = _lrelu(jnp.concatenate(pieces, axis=1))
    h = _lrelu(jnp.dot(mw2_ref[...], h,
                       preferred_element_type=jnp.float32
                       ).astype(jnp.bfloat16) + mb2_ref[...])
    h = _lrelu(jnp.dot(mw3_ref[...], h,
                       preferred_element_type=jnp.float32
                       ).astype(jnp.bfloat16) + mb3_ref[...])
    o = jax.nn.sigmoid(jnp.dot(mw4_ref[...], h,
                               preferred_element_type=jnp.float32) + mb4_ref[...])
    for d in range(_DC):
        out_ref[0, d:d + 1, :] = o[:, d * _P:(d + 1) * _P]


@jax.jit
def kernel(x, sw1, sb1, sw2, sb2, sw3, sb3,
           mw1, mb1, mw2, mb2, mw3, mb3, mw4, mb4):
    xr = x.reshape(_B, _D, _HW)

    def wspec(shape):
        return pl.BlockSpec(shape, lambda b, s, j: (0,) * len(shape))

    grid = (_B, _S, _NJ)
    out = pl.pallas_call(
        _body,
        grid=grid,
        in_specs=[
            pl.BlockSpec((1, _D, _P), lambda b, s, j: (b, 0, s)),
            wspec((64, 128)), wspec((64, 1)),
            wspec((32, 64)), wspec((32, 1)),
            wspec((16, 32)), wspec((16, 1)),
            wspec((32, 16)), wspec((32, 1)), wspec((32, 1)),
            wspec((32, 32)), wspec((32, 1)),
            wspec((16, 32)), wspec((16, 1)),
            wspec((1, 16)), wspec((1, 1)),
        ],
        out_specs=pl.BlockSpec((1, _DC, _P), lambda b, s, j: (b, j, s)),
        out_shape=jax.ShapeDtypeStruct((_B, _D, _HW), jnp.float32),
        scratch_shapes=[pltpu.VMEM((32, _P), jnp.float32)],
        compiler_params=pltpu.CompilerParams(
            dimension_semantics=("parallel", "parallel", "arbitrary"),
            vmem_limit_bytes=100 * 1024 * 1024,
        ),
    )(xr, sw1, sb1.reshape(64, 1), sw2, sb2.reshape(32, 1),
      sw3, sb3.reshape(16, 1),
      mw1[:, :16], mw1[:, 16:], mb1.reshape(32, 1),
      mw2.astype(jnp.bfloat16), mb2.reshape(32, 1).astype(jnp.bfloat16),
      mw3.astype(jnp.bfloat16), mb3.reshape(16, 1).astype(jnp.bfloat16),
      mw4.astype(jnp.bfloat16), mb4.reshape(1, 1))
    return out.reshape(_B, _D, _D, _D)


# speedup vs baseline: 4.5011x; 1.2287x over previous
import jax
import jax.numpy as jnp
from jax.experimental import pallas as pl
from jax.experimental.pallas import tpu as pltpu

_SLOPE = 0.01
_B = 4
_D = 128
_HW = _D * _D
_S = 2
_P = _HW // _S
_DC = 8
_NJ = _D // _DC
_CH = 256


def _lrelu(x):
    return jnp.maximum(x, _SLOPE * x)


def _body(x_ref, coef_ref, sw1_ref, sb1_ref, sw2_ref, sb2_ref, sw3_ref,
          sb3_ref, mw1f_ref, mb1_ref, bd2_ref, bb2_ref, bd3_ref, bb3_ref,
          bd4_ref, mb4_ref, out_ref, base_scr):
    j = pl.program_id(2)

    @pl.when(j == 0)
    def _stage():
        xb = x_ref[0]
        f = _lrelu(jnp.dot(sw1_ref[...], xb,
                           preferred_element_type=jnp.float32) + sb1_ref[...])
        f = _lrelu(jnp.dot(sw2_ref[...], f,
                           preferred_element_type=jnp.float32) + sb2_ref[...])
        f = jnp.dot(sw3_ref[...], f,
                    preferred_element_type=jnp.float32) + sb3_ref[...]
        base_scr[...] = jnp.dot(mw1f_ref[...], f,
                                preferred_element_type=jnp.float32) + mb1_ref[...]

    base = base_scr[...].astype(jnp.bfloat16)
    for k in range(8):
        coef = coef_ref[0, k]
        h = _lrelu(pltpu.repeat(base, _DC, axis=0) + coef)
        h = _lrelu(jnp.dot(bd2_ref[...], h,
                           preferred_element_type=jnp.float32
                           ).astype(jnp.bfloat16) + bb2_ref[...])
        h = _lrelu(jnp.dot(bd3_ref[...], h,
                           preferred_element_type=jnp.float32
                           ).astype(jnp.bfloat16) + bb3_ref[...])
        o = jax.nn.sigmoid(jnp.dot(bd4_ref[...], h,
                                   preferred_element_type=jnp.float32) + mb4_ref[...])
        out_ref[0, k * _DC:(k + 1) * _DC, :] = o


@jax.jit
def kernel(x, sw1, sb1, sw2, sb2, sw3, sb3,
           mw1, mb1, mw2, mb2, mw3, mb3, mw4, mb4):
    xr = x.reshape(_B, _D, _HW)
    bf = jnp.bfloat16

    pts = jnp.linspace(-1.0, 1.0, _D, dtype=jnp.float32)
    coef = (pts[:, None] * mw1[:, 16][None, :]).reshape(_NJ // 8, 8, _DC * 32, 1)

    eye = jnp.eye(_DC, dtype=jnp.float32)
    bd2 = jnp.kron(eye, mw2).astype(bf)
    bd3 = jnp.kron(eye, mw3).astype(bf)
    bd4 = jnp.kron(eye, mw4).astype(bf)
    bb2 = jnp.tile(mb2, _DC).reshape(_DC * 32, 1).astype(bf)
    bb3 = jnp.tile(mb3, _DC).reshape(_DC * 16, 1).astype(bf)

    def wspec(shape):
        return pl.BlockSpec(shape, lambda b, s, j: (0,) * len(shape))

    grid = (_B, _S, _NJ // 8)
    out = pl.pallas_call(
        _body,
        grid=grid,
        in_specs=[
            pl.BlockSpec((1, _D, _P), lambda b, s, j: (b, 0, s)),
            pl.BlockSpec((1, 8, _DC * 32, 1), lambda b, s, j: (j, 0, 0, 0)),
            wspec((64, 128)), wspec((64, 1)),
            wspec((32, 64)), wspec((32, 1)),
            wspec((16, 32)), wspec((16, 1)),
            wspec((32, 16)), wspec((32, 1)),
            wspec((_DC * 32, _DC * 32)), wspec((_DC * 32, 1)),
            wspec((_DC * 16, _DC * 32)), wspec((_DC * 16, 1)),
            wspec((_DC, _DC * 16)), wspec((1, 1)),
        ],
        out_specs=pl.BlockSpec((1, 8 * _DC, _P), lambda b, s, j: (b, j, s)),
        out_shape=jax.ShapeDtypeStruct((_B, _D, _HW), jnp.float32),
        scratch_shapes=[pltpu.VMEM((32, _P), jnp.float32)],
        compiler_params=pltpu.CompilerParams(
            dimension_semantics=("parallel", "parallel", "arbitrary"),
            vmem_limit_bytes=100 * 1024 * 1024,
        ),
    )(xr, coef.astype(bf), sw1, sb1.reshape(64, 1), sw2, sb2.reshape(32, 1),
      sw3, sb3.reshape(16, 1),
      mw1[:, :16], mb1.reshape(32, 1),
      bd2, bb2, bd3, bb3, bd4, mb4.reshape(1, 1))
    return out.reshape(_B, _D, _D, _D)
